# baseline (speedup 1.0000x reference)
import functools

import jax
import jax.numpy as jnp
from jax import lax
from jax.experimental import pallas as pl
from jax.experimental.pallas import tpu as pltpu
from jax.experimental.pallas import tpu_sc as plsc

N_NODES = 10000
IN_DIM = 256
HID_DIM = 256
ATT_DIM = 64
N_CLASSES = 5
N_TAILS = 1000
N_EDGES = 160000
N_AUG = N_NODES + N_TAILS + (N_CLASSES - 1)
NPAD = 11008
ROWS_PER_TILE = NPAD // 16
TPAD = 1024
NC, NS, LANES = 2, 16, 16
EDGES_PER_TILE = N_EDGES // NS
EB = 128
NB = 80
EPT = NB * EB
NBUF = 2
IBUF = 4

_HI = lax.Precision.HIGHEST


def _centers_body(lab_ref, f_ref, csum_ref, cnt_ref):
    i = pl.program_id(0)
    lab = lab_ref[0]
    cls = lax.broadcasted_iota(jnp.int32, (8, 1), 0) + 1
    oh = (lab == cls).astype(jnp.float32)
    psum = lax.dot_general(oh, f_ref[...], (((1,), (0,)), ((), ())),
                           precision=_HI, preferred_element_type=jnp.float32)
    pcnt = jnp.sum(oh, axis=1, keepdims=True)

    @pl.when(i == 0)
    def _():
        csum_ref[...] = jnp.zeros_like(csum_ref)
        cnt_ref[...] = jnp.zeros_like(cnt_ref)

    csum_ref[...] += psum
    cnt_ref[...] += jnp.broadcast_to(pcnt, cnt_ref.shape)


def _centers(features, labels):
    blk = 2000
    grid = N_NODES // blk
    lab3 = labels.reshape(grid, 1, blk)
    return pl.pallas_call(
        _centers_body,
        grid=(grid,),
        in_specs=[
            pl.BlockSpec((1, 1, blk), lambda i: (i, 0, 0)),
            pl.BlockSpec((blk, IN_DIM), lambda i: (i, 0)),
        ],
        out_specs=[
            pl.BlockSpec((8, IN_DIM), lambda i: (0, 0)),
            pl.BlockSpec((8, 128), lambda i: (0, 0)),
        ],
        out_shape=[
            jax.ShapeDtypeStruct((8, IN_DIM), jnp.float32),
            jax.ShapeDtypeStruct((8, 128), jnp.float32),
        ],
    )(lab3, features)


def _ctrmap_body(lab_ref, csum_ref, cnt_ref, ctr_ref, map_ref):
    i = pl.program_id(0)
    cnt = jnp.maximum(cnt_ref[:, 0:1], 1.0)
    centers = csum_ref[...] / cnt

    @pl.when(i == 0)
    def _():
        ctr_ref[...] = centers

    lab = lab_ref[0, 0, :]
    cidx = (lab[:, None] + 3) % 4
    jcol = lax.broadcasted_iota(jnp.int32, (cidx.shape[0], 8), 1)
    oh = (cidx == jcol).astype(jnp.float32)
    map_ref[...] = lax.dot_general(oh, centers, (((1,), (0,)), ((), ())),
                                   precision=_HI,
                                   preferred_element_type=jnp.float32)


def _ctrmap(labels, csum, cnt):
    blk = 2000
    grid = N_NODES // blk
    lab3 = labels.reshape(grid, 1, blk)
    return pl.pallas_call(
        _ctrmap_body,
        grid=(grid,),
        in_specs=[
            pl.BlockSpec((1, 1, blk), lambda i: (i, 0, 0)),
            pl.BlockSpec((8, IN_DIM), lambda i: (0, 0)),
            pl.BlockSpec((8, 128), lambda i: (0, 0)),
        ],
        out_specs=[
            pl.BlockSpec((8, IN_DIM), lambda i: (0, 0)),
            pl.BlockSpec((blk, IN_DIM), lambda i: (i, 0)),
        ],
        out_shape=[
            jax.ShapeDtypeStruct((8, IN_DIM), jnp.float32),
            jax.ShapeDtypeStruct((N_NODES, IN_DIM), jnp.float32),
        ],
    )(lab3, csum, cnt)


def _tails_sc(features, ctrmap, ch0, ch1):
    rows_w = TPAD // (NC * NS)
    mesh = plsc.VectorSubcoreMesh(core_axis_name="c", subcore_axis_name="s")

    @functools.partial(
        pl.kernel,
        out_type=[
            jax.ShapeDtypeStruct((TPAD, IN_DIM), jnp.float32),
            jax.ShapeDtypeStruct((TPAD, IN_DIM), jnp.float32),
            jax.ShapeDtypeStruct((TPAD, IN_DIM), jnp.float32),
            jax.ShapeDtypeStruct((TPAD, IN_DIM), jnp.float32),
        ],
        mesh=mesh,
        scratch_types=[
            pltpu.VMEM((rows_w,), jnp.int32),
            pltpu.VMEM((rows_w, IN_DIM), jnp.float32),
            pltpu.SemaphoreType.DMA,
        ],
    )
    def k(f_hbm, m_hbm, ch0_hbm, ch1_hbm, t0_hbm, t1_hbm, c0_hbm, c1_hbm,
          idx_v, rows_v, sem):
        cid = lax.axis_index("c")
        sid = lax.axis_index("s")
        wid = sid * NC + cid
        base = wid * rows_w
        for ch_hbm, t_hbm, c_hbm in ((ch0_hbm, t0_hbm, c0_hbm),
                                     (ch1_hbm, t1_hbm, c1_hbm)):
            pltpu.sync_copy(ch_hbm.at[pl.ds(base, rows_w)], idx_v)
            pltpu.async_copy(f_hbm.at[idx_v], rows_v, sem).wait()
            pltpu.sync_copy(rows_v, t_hbm.at[pl.ds(base, rows_w)])
            pltpu.async_copy(m_hbm.at[idx_v], rows_v, sem).wait()
            pltpu.sync_copy(rows_v, c_hbm.at[pl.ds(base, rows_w)])

    return k(features, ctrmap, ch0, ch1)


def _smote_body(t0_ref, t1_ref, c0_ref, c1_ref, smw_ref, new0_ref, new1_ref):
    smw = smw_ref[...]
    for t_ref, c_ref, o_ref in ((t0_ref, c0_ref, new0_ref),
                                (t1_ref, c1_ref, new1_ref)):
        tail = t_ref[...]
        o_ref[...] = tail + lax.dot_general(
            tail - c_ref[...], smw, (((1,), (0,)), ((), ())),
            precision=_HI, preferred_element_type=jnp.float32)


def _smote(t0, t1, c0, c1, sm_w):
    return pl.pallas_call(
        _smote_body,
        out_shape=[
            jax.ShapeDtypeStruct((TPAD, IN_DIM), jnp.float32),
            jax.ShapeDtypeStruct((TPAD, IN_DIM), jnp.float32),
        ],
    )(t0, t1, c0, c1, sm_w)


def _previdx_body(s0_ref, s1_ref, g10_ref, g11_ref, g20_ref, g21_ref):
    s0 = s0_ref[...]
    s1 = s1_ref[...]
    r0 = s0 + jnp.where(s0 >= N_NODES + N_TAILS, 1000, 0)
    r1 = s1 + jnp.where(s1 >= N_NODES, 1000, 0)
    for c in (0, 1):
        g10_ref[c] = r0 * 2 + c
        g11_ref[c] = r1 * 2 + c
        g20_ref[c] = s0 * 2 + c
        g21_ref[c] = s1 * 2 + c


def _previdx(src0, src1):
    rows = N_EDGES // 128
    sh = jax.ShapeDtypeStruct((2, rows, 128), jnp.int32)
    return pl.pallas_call(
        _previdx_body,
        out_shape=[sh, sh, sh, sh],
    )(src0.reshape(rows, 128), src1.reshape(rows, 128))


def _pack_idx(g, d):
    g3 = g.reshape(2, NS, EDGES_PER_TILE)
    gp = jnp.zeros((2, NS, EPT - EDGES_PER_TILE), jnp.int32)
    g4 = jnp.concatenate([g3, gp], axis=2).reshape(2, NS, NB, EB)
    d2 = d.reshape(NS, EDGES_PER_TILE)
    dp = jnp.full((NS, EPT - EDGES_PER_TILE), N_AUG, jnp.int32)
    d3 = jnp.concatenate([d2, dp], axis=1).reshape(NS, NB, EB)
    d4 = jnp.broadcast_to(d3[None], (2, NS, NB, EB))
    return jnp.stack([g4, d4], axis=3)


def _scatter_sc(x2, idx, zeros):
    mesh = plsc.VectorSubcoreMesh(core_axis_name="c", subcore_axis_name="s")

    @functools.partial(
        pl.kernel,
        out_type=jax.ShapeDtypeStruct((NC, NPAD, 128), jnp.float32),
        mesh=mesh,
        scratch_types=[
            pltpu.VMEM((IBUF, 2, EB), jnp.int32),
            pltpu.VMEM((NBUF, EB, 128), jnp.float32),
            pltpu.VMEM_SHARED((NPAD, 128), jnp.float32),
            pltpu.SemaphoreType.DMA,
            pltpu.SemaphoreType.DMA,
            pltpu.SemaphoreType.DMA,
            pltpu.SemaphoreType.DMA,
            pltpu.SemaphoreType.DMA,
            pltpu.SemaphoreType.DMA,
        ],
    )
    def k(x2_hbm, i_hbm, z_hbm, out_hbm,
          idx_v, rows_v, acc_sh, is0, is1, is2, is3, gs0, gs1):
        cid = lax.axis_index("c")
        sid = lax.axis_index("s")
        isems = (is0, is1, is2, is3)
        gsems = (gs0, gs1)

        for i in range(IBUF):
            pltpu.async_copy(i_hbm.at[cid, sid, i], idx_v.at[i], isems[i])
        for b in range(NBUF):
            pltpu.make_async_copy(
                i_hbm.at[cid, sid, b], idx_v.at[b], isems[b]).wait()
            pltpu.async_copy(x2_hbm.at[idx_v.at[b, 0]], rows_v.at[b],
                             gsems[b])
        pltpu.sync_copy(z_hbm, acc_sh.at[pl.ds(sid * ROWS_PER_TILE,
                                               ROWS_PER_TILE)])
        plsc.subcore_barrier()

        def body(g, carry):
            for b4 in range(IBUF):
                j = g * IBUF + b4
                rb = b4 % NBUF
                ib2 = (b4 + NBUF) % IBUF
                pltpu.make_async_copy(
                    x2_hbm.at[idx_v.at[b4, 0]], rows_v.at[rb],
                    gsems[rb]).wait()
                pltpu.sync_copy(rows_v.at[rb], acc_sh.at[idx_v.at[b4, 1]],
                                add=True)

                @pl.when(j + IBUF < NB)
                def _():
                    pltpu.async_copy(i_hbm.at[cid, sid, j + IBUF],
                                     idx_v.at[b4], isems[b4])

                @pl.when(j + NBUF < NB)
                def _():
                    pltpu.make_async_copy(
                        i_hbm.at[cid, sid, j + NBUF], idx_v.at[ib2],
                        isems[ib2]).wait()
                    pltpu.async_copy(x2_hbm.at[idx_v.at[ib2, 0]],
                                     rows_v.at[rb], gsems[rb])
            return carry

        lax.fori_loop(0, NB // IBUF, body, 0)
        plsc.subcore_barrier()
        pltpu.sync_copy(
            acc_sh.at[pl.ds(sid * ROWS_PER_TILE, ROWS_PER_TILE)],
            out_hbm.at[cid, pl.ds(sid * ROWS_PER_TILE, ROWS_PER_TILE)])

    return k(x2, idx, zeros)


def _layer_body(relu, a_ref, w_ref, b_ref, o_ref):
    w = w_ref[...]
    h = (lax.dot_general(a_ref[0], w[:128], (((1,), (0,)), ((), ())),
                         precision=_HI, preferred_element_type=jnp.float32)
         + lax.dot_general(a_ref[1], w[128:], (((1,), (0,)), ((), ())),
                           precision=_HI, preferred_element_type=jnp.float32)
         + b_ref[...])
    o_ref[...] = jnp.maximum(h, 0.0) if relu else h


def _layer(agg, W, b, relu):
    blk = 1376
    grid = NPAD // blk
    return pl.pallas_call(
        functools.partial(_layer_body, relu),
        grid=(grid,),
        in_specs=[
            pl.BlockSpec((2, blk, 128), lambda i: (0, i, 0)),
            pl.BlockSpec((256, HID_DIM), lambda i: (0, 0)),
            pl.BlockSpec((1, HID_DIM), lambda i: (0, 0)),
        ],
        out_specs=pl.BlockSpec((blk, HID_DIM), lambda i: (i, 0)),
        out_shape=jax.ShapeDtypeStruct((NPAD, HID_DIM), jnp.float32),
    )(agg, W, b[None, :])


def _final_body(a0_ref, a1_ref, w2_ref, b2_ref, saw_ref, sab_ref, saa_ref,
                e0_ref, e1_ref, out_ref):
    w2 = w2_ref[...]
    b2 = b2_ref[...]

    def dense(a_ref):
        return (lax.dot_general(a_ref[0], w2[:128], (((1,), (0,)), ((), ())),
                                precision=_HI,
                                preferred_element_type=jnp.float32)
                + lax.dot_general(a_ref[1], w2[128:], (((1,), (0,)), ((), ())),
                                  precision=_HI,
                                  preferred_element_type=jnp.float32)
                + b2)

    e0 = dense(a0_ref)
    e1 = dense(a1_ref)
    e0_ref[...] = e0
    e1_ref[...] = e1
    saw = saw_ref[...]
    sab = sab_ref[...]
    saa = saa_ref[...]

    def score(e):
        t = jnp.tanh(lax.dot_general(e, saw, (((1,), (0,)), ((), ())),
                                     precision=_HI,
                                     preferred_element_type=jnp.float32) + sab)
        return jnp.sum(t * saa, axis=1, keepdims=True)

    w0 = score(e0)
    w1 = score(e1)
    m = jnp.maximum(w0, w1)
    x0 = jnp.exp(w0 - m)
    x1 = jnp.exp(w1 - m)
    inv = 1.0 / (x0 + x1)
    out_ref[...] = (x0 * e0 + x1 * e1) * inv


def _final(agg0, agg1, W2, b2, sa_w, sa_b, sa_a):
    blk = 1376
    grid = NPAD // blk
    return pl.pallas_call(
        _final_body,
        grid=(grid,),
        in_specs=[
            pl.BlockSpec((2, blk, 128), lambda i: (0, i, 0)),
            pl.BlockSpec((2, blk, 128), lambda i: (0, i, 0)),
            pl.BlockSpec((256, HID_DIM), lambda i: (0, 0)),
            pl.BlockSpec((1, HID_DIM), lambda i: (0, 0)),
            pl.BlockSpec((HID_DIM, ATT_DIM), lambda i: (0, 0)),
            pl.BlockSpec((1, ATT_DIM), lambda i: (0, 0)),
            pl.BlockSpec((1, ATT_DIM), lambda i: (0, 0)),
        ],
        out_specs=[
            pl.BlockSpec((blk, HID_DIM), lambda i: (i, 0)),
            pl.BlockSpec((blk, HID_DIM), lambda i: (i, 0)),
            pl.BlockSpec((blk, HID_DIM), lambda i: (i, 0)),
        ],
        out_shape=[
            jax.ShapeDtypeStruct((NPAD, HID_DIM), jnp.float32),
            jax.ShapeDtypeStruct((NPAD, HID_DIM), jnp.float32),
            jax.ShapeDtypeStruct((NPAD, HID_DIM), jnp.float32),
        ],
    )(agg0, agg1, W2, b2[None, :], sa_w, sa_b[None, :], sa_a.reshape(1, ATT_DIM))


def kernel(features, labels, chosen_tails_0, chosen_tails_1, edge_index_0,
           edge_index_1, sm_weight_center, W1, b1, W2, b2, sa_w, sa_b, sa_a):
    i32 = jnp.int32
    labels = labels.astype(i32)
    pad = jnp.zeros((TPAD - N_TAILS,), i32)
    ch0 = jnp.concatenate([chosen_tails_0.astype(i32), pad])
    ch1 = jnp.concatenate([chosen_tails_1.astype(i32), pad])
    src0, dst0 = edge_index_0[0].astype(i32), edge_index_0[1].astype(i32)
    src1, dst1 = edge_index_1[0].astype(i32), edge_index_1[1].astype(i32)
    zeros = jnp.zeros((ROWS_PER_TILE, 128), jnp.float32)

    g10, g11, g20, g21 = _previdx(src0, src1)
    i10 = _pack_idx(g10.reshape(2, N_EDGES), dst0)
    i11 = _pack_idx(g11.reshape(2, N_EDGES), dst1)
    i20 = _pack_idx(g20.reshape(2, N_EDGES), dst0)
    i21 = _pack_idx(g21.reshape(2, N_EDGES), dst1)

    csum, cnt = _centers(features, labels)
    ctr, ctrmap = _ctrmap(labels, csum, cnt)
    t0, t1, c0, c1 = _tails_sc(features, ctrmap, ch0, ch1)
    new0, new1 = _smote(t0, t1, c0, c1, sm_weight_center)

    x = jnp.concatenate([features, new0[:N_TAILS], new1[:N_TAILS], ctr[:4]],
                        axis=0)
    x2 = x.reshape(-1, 128)

    agg10 = _scatter_sc(x2, i10, zeros)
    agg11 = _scatter_sc(x2, i11, zeros)
    h0 = _layer(agg10, W1, b1, relu=True)
    h1 = _layer(agg11, W1, b1, relu=True)

    agg20 = _scatter_sc(h0.reshape(-1, 128), i20, zeros)
    agg21 = _scatter_sc(h1.reshape(-1, 128), i21, zeros)
    e0, e1, outp = _final(agg20, agg21, W2, b2, sa_w, sa_b, sa_a)

    return (e0[:N_AUG], e1[:N_AUG], outp[:N_NODES])


# speedup vs baseline: 2.1166x; 2.1166x over previous
import functools

import jax
import jax.numpy as jnp
from jax import lax
from jax.experimental import pallas as pl
from jax.experimental.pallas import tpu as pltpu
from jax.experimental.pallas import tpu_sc as plsc

N_NODES = 10000
IN_DIM = 256
HID_DIM = 256
ATT_DIM = 64
N_CLASSES = 5
N_TAILS = 1000
N_EDGES = 160000
N_AUG = N_NODES + N_TAILS + (N_CLASSES - 1)
NPAD = 11008
ROWS_PER_TILE = NPAD // 16
TPAD = 1024
NC, NS, LANES = 2, 16, 16
EDGES_PER_TILE = N_EDGES // NS
EB = 80
NB = 126
EPT = NB * EB
NBUF = 2
IBUF = 3

_HI = lax.Precision.HIGHEST


def _centers_body(lab_ref, f_ref, csum_ref, cnt_ref):
    i = pl.program_id(0)
    lab = lab_ref[0]
    cls = lax.broadcasted_iota(jnp.int32, (8, 1), 0) + 1
    oh = (lab == cls).astype(jnp.float32)
    psum = lax.dot_general(oh, f_ref[...], (((1,), (0,)), ((), ())),
                           precision=_HI, preferred_element_type=jnp.float32)
    pcnt = jnp.sum(oh, axis=1, keepdims=True)

    @pl.when(i == 0)
    def _():
        csum_ref[...] = jnp.zeros_like(csum_ref)
        cnt_ref[...] = jnp.zeros_like(cnt_ref)

    csum_ref[...] += psum
    cnt_ref[...] += jnp.broadcast_to(pcnt, cnt_ref.shape)


def _centers(features, labels):
    blk = 2000
    grid = N_NODES // blk
    lab3 = labels.reshape(grid, 1, blk)
    return pl.pallas_call(
        _centers_body,
        grid=(grid,),
        in_specs=[
            pl.BlockSpec((1, 1, blk), lambda i: (i, 0, 0)),
            pl.BlockSpec((blk, IN_DIM), lambda i: (i, 0)),
        ],
        out_specs=[
            pl.BlockSpec((8, IN_DIM), lambda i: (0, 0)),
            pl.BlockSpec((8, 128), lambda i: (0, 0)),
        ],
        out_shape=[
            jax.ShapeDtypeStruct((8, IN_DIM), jnp.float32),
            jax.ShapeDtypeStruct((8, 128), jnp.float32),
        ],
    )(lab3, features)


def _ctrmap_body(lab_ref, csum_ref, cnt_ref, ctr_ref, map_ref):
    i = pl.program_id(0)
    cnt = jnp.maximum(cnt_ref[:, 0:1], 1.0)
    centers = csum_ref[...] / cnt

    @pl.when(i == 0)
    def _():
        ctr_ref[...] = centers

    lab = lab_ref[0, 0, :]
    cidx = (lab[:, None] + 3) % 4
    jcol = lax.broadcasted_iota(jnp.int32, (cidx.shape[0], 8), 1)
    oh = (cidx == jcol).astype(jnp.float32)
    map_ref[...] = lax.dot_general(oh, centers, (((1,), (0,)), ((), ())),
                                   precision=_HI,
                                   preferred_element_type=jnp.float32)


def _ctrmap(labels, csum, cnt):
    blk = 2000
    grid = N_NODES // blk
    lab3 = labels.reshape(grid, 1, blk)
    return pl.pallas_call(
        _ctrmap_body,
        grid=(grid,),
        in_specs=[
            pl.BlockSpec((1, 1, blk), lambda i: (i, 0, 0)),
            pl.BlockSpec((8, IN_DIM), lambda i: (0, 0)),
            pl.BlockSpec((8, 128), lambda i: (0, 0)),
        ],
        out_specs=[
            pl.BlockSpec((8, IN_DIM), lambda i: (0, 0)),
            pl.BlockSpec((blk, IN_DIM), lambda i: (i, 0)),
        ],
        out_shape=[
            jax.ShapeDtypeStruct((8, IN_DIM), jnp.float32),
            jax.ShapeDtypeStruct((N_NODES, IN_DIM), jnp.float32),
        ],
    )(lab3, csum, cnt)


def _tails_sc(features, ctrmap, ch0, ch1):
    rows_w = TPAD // (NC * NS)
    mesh = plsc.VectorSubcoreMesh(core_axis_name="c", subcore_axis_name="s")

    @functools.partial(
        pl.kernel,
        out_type=[
            jax.ShapeDtypeStruct((TPAD, IN_DIM), jnp.float32),
            jax.ShapeDtypeStruct((TPAD, IN_DIM), jnp.float32),
            jax.ShapeDtypeStruct((TPAD, IN_DIM), jnp.float32),
            jax.ShapeDtypeStruct((TPAD, IN_DIM), jnp.float32),
        ],
        mesh=mesh,
        scratch_types=[
            pltpu.VMEM((rows_w,), jnp.int32),
            pltpu.VMEM((rows_w, IN_DIM), jnp.float32),
            pltpu.SemaphoreType.DMA,
        ],
    )
    def k(f_hbm, m_hbm, ch0_hbm, ch1_hbm, t0_hbm, t1_hbm, c0_hbm, c1_hbm,
          idx_v, rows_v, sem):
        cid = lax.axis_index("c")
        sid = lax.axis_index("s")
        wid = sid * NC + cid
        base = wid * rows_w
        for ch_hbm, t_hbm, c_hbm in ((ch0_hbm, t0_hbm, c0_hbm),
                                     (ch1_hbm, t1_hbm, c1_hbm)):
            pltpu.sync_copy(ch_hbm.at[pl.ds(base, rows_w)], idx_v)
            pltpu.async_copy(f_hbm.at[idx_v], rows_v, sem).wait()
            pltpu.sync_copy(rows_v, t_hbm.at[pl.ds(base, rows_w)])
            pltpu.async_copy(m_hbm.at[idx_v], rows_v, sem).wait()
            pltpu.sync_copy(rows_v, c_hbm.at[pl.ds(base, rows_w)])

    return k(features, ctrmap, ch0, ch1)


def _smote_body(t0_ref, t1_ref, c0_ref, c1_ref, smw_ref, new0_ref, new1_ref):
    smw = smw_ref[...]
    for t_ref, c_ref, o_ref in ((t0_ref, c0_ref, new0_ref),
                                (t1_ref, c1_ref, new1_ref)):
        tail = t_ref[...]
        o_ref[...] = tail + lax.dot_general(
            tail - c_ref[...], smw, (((1,), (0,)), ((), ())),
            precision=_HI, preferred_element_type=jnp.float32)


def _smote(t0, t1, c0, c1, sm_w):
    return pl.pallas_call(
        _smote_body,
        out_shape=[
            jax.ShapeDtypeStruct((TPAD, IN_DIM), jnp.float32),
            jax.ShapeDtypeStruct((TPAD, IN_DIM), jnp.float32),
        ],
    )(t0, t1, c0, c1, sm_w)


def _pad_edges(src, dst):
    s2 = src.reshape(NS, EDGES_PER_TILE)
    sp = jnp.zeros((NS, EPT - EDGES_PER_TILE), jnp.int32)
    d2 = dst.reshape(NS, EDGES_PER_TILE)
    dp = jnp.full((NS, EPT - EDGES_PER_TILE), N_AUG, jnp.int32)
    return (jnp.concatenate([s2, sp], axis=1).reshape(-1),
            jnp.concatenate([d2, dp], axis=1).reshape(-1))


def _scatter_sc(x2, src, dst, zeros, thresh):
    mesh = plsc.VectorSubcoreMesh(core_axis_name="c", subcore_axis_name="s")

    @functools.partial(
        pl.kernel,
        out_type=jax.ShapeDtypeStruct((NC, NPAD, 128), jnp.float32),
        mesh=mesh,
        scratch_types=[
            pltpu.VMEM((EPT,), jnp.int32),
            pltpu.VMEM((EPT,), jnp.int32),
            pltpu.VMEM((IBUF, EB), jnp.int32),
            pltpu.VMEM((IBUF, EB), jnp.int32),
            pltpu.VMEM((NBUF, EB, 128), jnp.float32),
            pltpu.VMEM_SHARED((NPAD, 128), jnp.float32),
            pltpu.SemaphoreType.DMA,
            pltpu.SemaphoreType.DMA,
        ],
    )
    def k(x2_hbm, src_hbm, dst_hbm, z_hbm, out_hbm,
          src_v, dst_v, gidx_v, didx_v, rows_v, acc_sh, gs0, gs1):
        cid = lax.axis_index("c")
        sid = lax.axis_index("s")
        gsems = (gs0, gs1)
        ebase = sid * EPT
        pltpu.sync_copy(src_hbm.at[pl.ds(ebase, EPT)], src_v)
        pltpu.sync_copy(dst_hbm.at[pl.ds(ebase, EPT)], dst_v)

        def compute_idx(j, slot):
            off = j * EB
            for q in range(EB // LANES):
                s = src_v[pl.ds(off + q * LANES, LANES)]
                s = s + jnp.where(s >= thresh, 1000, 0)
                gidx_v[slot, pl.ds(q * LANES, LANES)] = s * 2 + cid
                d = dst_v[pl.ds(off + q * LANES, LANES)]
                didx_v[slot, pl.ds(q * LANES, LANES)] = d

        for b in range(NBUF):
            compute_idx(b, b)
            pltpu.async_copy(x2_hbm.at[gidx_v.at[b]], rows_v.at[b],
                             gsems[b])
        pltpu.sync_copy(z_hbm, acc_sh.at[pl.ds(sid * ROWS_PER_TILE,
                                               ROWS_PER_TILE)])
        plsc.subcore_barrier()

        def body(g, carry):
            for b6 in range(6):
                j = g * 6 + b6
                ib = b6 % IBUF
                ib2 = (b6 + NBUF) % IBUF
                rb = b6 % NBUF
                @pl.when(j + NBUF < NB)
                def _():
                    compute_idx(j + NBUF, ib2)

                pltpu.make_async_copy(
                    x2_hbm.at[gidx_v.at[ib]], rows_v.at[rb],
                    gsems[rb]).wait()
                pltpu.sync_copy(rows_v.at[rb], acc_sh.at[didx_v.at[ib]],
                                add=True)

                @pl.when(j + NBUF < NB)
                def _():
                    pltpu.async_copy(x2_hbm.at[gidx_v.at[ib2]],
                                     rows_v.at[rb], gsems[rb])
            return carry

        lax.fori_loop(0, NB // 6, body, 0)
        plsc.subcore_barrier()
        pltpu.sync_copy(
            acc_sh.at[pl.ds(sid * ROWS_PER_TILE, ROWS_PER_TILE)],
            out_hbm.at[cid, pl.ds(sid * ROWS_PER_TILE, ROWS_PER_TILE)])

    return k(x2, src, dst, zeros)


def _layer_body(relu, a_ref, w_ref, b_ref, o_ref):
    w = w_ref[...]
    h = (lax.dot_general(a_ref[0], w[:128], (((1,), (0,)), ((), ())),
                         precision=_HI, preferred_element_type=jnp.float32)
         + lax.dot_general(a_ref[1], w[128:], (((1,), (0,)), ((), ())),
                           precision=_HI, preferred_element_type=jnp.float32)
         + b_ref[...])
    o_ref[...] = jnp.maximum(h, 0.0) if relu else h


def _layer(agg, W, b, relu):
    blk = 1376
    grid = NPAD // blk
    return pl.pallas_call(
        functools.partial(_layer_body, relu),
        grid=(grid,),
        in_specs=[
            pl.BlockSpec((2, blk, 128), lambda i: (0, i, 0)),
            pl.BlockSpec((256, HID_DIM), lambda i: (0, 0)),
            pl.BlockSpec((1, HID_DIM), lambda i: (0, 0)),
        ],
        out_specs=pl.BlockSpec((blk, HID_DIM), lambda i: (i, 0)),
        out_shape=jax.ShapeDtypeStruct((NPAD, HID_DIM), jnp.float32),
    )(agg, W, b[None, :])


def _final_body(a0_ref, a1_ref, w2_ref, b2_ref, saw_ref, sab_ref, saa_ref,
                e0_ref, e1_ref, out_ref):
    w2 = w2_ref[...]
    b2 = b2_ref[...]

    def dense(a_ref):
        return (lax.dot_general(a_ref[0], w2[:128], (((1,), (0,)), ((), ())),
                                precision=_HI,
                                preferred_element_type=jnp.float32)
                + lax.dot_general(a_ref[1], w2[128:], (((1,), (0,)), ((), ())),
                                  precision=_HI,
                                  preferred_element_type=jnp.float32)
                + b2)

    e0 = dense(a0_ref)
    e1 = dense(a1_ref)
    e0_ref[...] = e0
    e1_ref[...] = e1
    saw = saw_ref[...]
    sab = sab_ref[...]
    saa = saa_ref[...]

    def score(e):
        t = jnp.tanh(lax.dot_general(e, saw, (((1,), (0,)), ((), ())),
                                     precision=_HI,
                                     preferred_element_type=jnp.float32) + sab)
        return jnp.sum(t * saa, axis=1, keepdims=True)

    w0 = score(e0)
    w1 = score(e1)
    m = jnp.maximum(w0, w1)
    x0 = jnp.exp(w0 - m)
    x1 = jnp.exp(w1 - m)
    inv = 1.0 / (x0 + x1)
    out_ref[...] = (x0 * e0 + x1 * e1) * inv


def _final(agg0, agg1, W2, b2, sa_w, sa_b, sa_a):
    blk = 1376
    grid = NPAD // blk
    return pl.pallas_call(
        _final_body,
        grid=(grid,),
        in_specs=[
            pl.BlockSpec((2, blk, 128), lambda i: (0, i, 0)),
            pl.BlockSpec((2, blk, 128), lambda i: (0, i, 0)),
            pl.BlockSpec((256, HID_DIM), lambda i: (0, 0)),
            pl.BlockSpec((1, HID_DIM), lambda i: (0, 0)),
            pl.BlockSpec((HID_DIM, ATT_DIM), lambda i: (0, 0)),
            pl.BlockSpec((1, ATT_DIM), lambda i: (0, 0)),
            pl.BlockSpec((1, ATT_DIM), lambda i: (0, 0)),
        ],
        out_specs=[
            pl.BlockSpec((blk, HID_DIM), lambda i: (i, 0)),
            pl.BlockSpec((blk, HID_DIM), lambda i: (i, 0)),
            pl.BlockSpec((blk, HID_DIM), lambda i: (i, 0)),
        ],
        out_shape=[
            jax.ShapeDtypeStruct((NPAD, HID_DIM), jnp.float32),
            jax.ShapeDtypeStruct((NPAD, HID_DIM), jnp.float32),
            jax.ShapeDtypeStruct((NPAD, HID_DIM), jnp.float32),
        ],
    )(agg0, agg1, W2, b2[None, :], sa_w, sa_b[None, :], sa_a.reshape(1, ATT_DIM))


def kernel(features, labels, chosen_tails_0, chosen_tails_1, edge_index_0,
           edge_index_1, sm_weight_center, W1, b1, W2, b2, sa_w, sa_b, sa_a):
    i32 = jnp.int32
    labels = labels.astype(i32)
    pad = jnp.zeros((TPAD - N_TAILS,), i32)
    ch0 = jnp.concatenate([chosen_tails_0.astype(i32), pad])
    ch1 = jnp.concatenate([chosen_tails_1.astype(i32), pad])
    src0, dst0 = edge_index_0[0].astype(i32), edge_index_0[1].astype(i32)
    src1, dst1 = edge_index_1[0].astype(i32), edge_index_1[1].astype(i32)
    zeros = jnp.zeros((ROWS_PER_TILE, 128), jnp.float32)

    src0, dst0 = _pad_edges(src0, dst0)
    src1, dst1 = _pad_edges(src1, dst1)

    csum, cnt = _centers(features, labels)
    ctr, ctrmap = _ctrmap(labels, csum, cnt)
    t0, t1, c0, c1 = _tails_sc(features, ctrmap, ch0, ch1)
    new0, new1 = _smote(t0, t1, c0, c1, sm_weight_center)

    x = jnp.concatenate([features, new0[:N_TAILS], new1[:N_TAILS], ctr[:4]],
                        axis=0)
    x2 = x.reshape(-1, 128)

    agg10 = _scatter_sc(x2, src0, dst0, zeros, N_NODES + N_TAILS)
    agg11 = _scatter_sc(x2, src1, dst1, zeros, N_NODES)
    h0 = _layer(agg10, W1, b1, relu=True)
    h1 = _layer(agg11, W1, b1, relu=True)

    big = 2 ** 30
    agg20 = _scatter_sc(h0.reshape(-1, 128), src0, dst0, zeros, big)
    agg21 = _scatter_sc(h1.reshape(-1, 128), src1, dst1, zeros, big)
    e0, e1, outp = _final(agg20, agg21, W2, b2, sa_w, sa_b, sa_a)

    return (e0[:N_AUG], e1[:N_AUG], outp[:N_NODES])


# speedup vs baseline: 2.1597x; 1.0203x over previous
import functools

import jax
import jax.numpy as jnp
from jax import lax
from jax.experimental import pallas as pl
from jax.experimental.pallas import tpu as pltpu
from jax.experimental.pallas import tpu_sc as plsc

N_NODES = 10000
IN_DIM = 256
HID_DIM = 256
ATT_DIM = 64
N_CLASSES = 5
N_TAILS = 1000
N_EDGES = 160000
N_AUG = N_NODES + N_TAILS + (N_CLASSES - 1)
NPAD = 11008
ROWS_PER_TILE = NPAD // 16
TPAD = 1024
NC, NS, LANES = 2, 16, 16
EDGES_PER_TILE = N_EDGES // NS
EB = 80
NB = 126
EPT = NB * EB
NBUF = 2
IBUF = 3

_HI = lax.Precision.HIGHEST


def _dot3(a, w):
    ah = a.astype(jnp.bfloat16)
    al = (a - ah.astype(jnp.float32)).astype(jnp.bfloat16)
    wh = w.astype(jnp.bfloat16)
    wl = (w - wh.astype(jnp.float32)).astype(jnp.bfloat16)

    def d(x, y):
        return lax.dot_general(x, y, (((1,), (0,)), ((), ())),
                               preferred_element_type=jnp.float32)

    return d(ah, wh) + d(ah, wl) + d(al, wh)


def _centers_body(lab_ref, f_ref, csum_ref, cnt_ref):
    i = pl.program_id(0)
    lab = lab_ref[0]
    cls = lax.broadcasted_iota(jnp.int32, (8, 1), 0) + 1
    oh = (lab == cls).astype(jnp.float32)
    psum = lax.dot_general(oh, f_ref[...], (((1,), (0,)), ((), ())),
                           precision=_HI, preferred_element_type=jnp.float32)
    pcnt = jnp.sum(oh, axis=1, keepdims=True)

    @pl.when(i == 0)
    def _():
        csum_ref[...] = jnp.zeros_like(csum_ref)
        cnt_ref[...] = jnp.zeros_like(cnt_ref)

    csum_ref[...] += psum
    cnt_ref[...] += jnp.broadcast_to(pcnt, cnt_ref.shape)


def _centers(features, labels):
    blk = 2000
    grid = N_NODES // blk
    lab3 = labels.reshape(grid, 1, blk)
    return pl.pallas_call(
        _centers_body,
        grid=(grid,),
        in_specs=[
            pl.BlockSpec((1, 1, blk), lambda i: (i, 0, 0)),
            pl.BlockSpec((blk, IN_DIM), lambda i: (i, 0)),
        ],
        out_specs=[
            pl.BlockSpec((8, IN_DIM), lambda i: (0, 0)),
            pl.BlockSpec((8, 128), lambda i: (0, 0)),
        ],
        out_shape=[
            jax.ShapeDtypeStruct((8, IN_DIM), jnp.float32),
            jax.ShapeDtypeStruct((8, 128), jnp.float32),
        ],
    )(lab3, features)


def _ctrmap_body(lab_ref, csum_ref, cnt_ref, ctr_ref, map_ref):
    i = pl.program_id(0)
    cnt = jnp.maximum(cnt_ref[:, 0:1], 1.0)
    centers = csum_ref[...] / cnt

    @pl.when(i == 0)
    def _():
        ctr_ref[...] = centers

    lab = lab_ref[0, 0, :]
    cidx = (lab[:, None] + 3) % 4
    jcol = lax.broadcasted_iota(jnp.int32, (cidx.shape[0], 8), 1)
    oh = (cidx == jcol).astype(jnp.float32)
    map_ref[...] = lax.dot_general(oh, centers, (((1,), (0,)), ((), ())),
                                   precision=_HI,
                                   preferred_element_type=jnp.float32)


def _ctrmap(labels, csum, cnt):
    blk = 2000
    grid = N_NODES // blk
    lab3 = labels.reshape(grid, 1, blk)
    return pl.pallas_call(
        _ctrmap_body,
        grid=(grid,),
        in_specs=[
            pl.BlockSpec((1, 1, blk), lambda i: (i, 0, 0)),
            pl.BlockSpec((8, IN_DIM), lambda i: (0, 0)),
            pl.BlockSpec((8, 128), lambda i: (0, 0)),
        ],
        out_specs=[
            pl.BlockSpec((8, IN_DIM), lambda i: (0, 0)),
            pl.BlockSpec((blk, IN_DIM), lambda i: (i, 0)),
        ],
        out_shape=[
            jax.ShapeDtypeStruct((8, IN_DIM), jnp.float32),
            jax.ShapeDtypeStruct((N_NODES, IN_DIM), jnp.float32),
        ],
    )(lab3, csum, cnt)


def _tails_sc(features, ctrmap, ch0, ch1):
    rows_w = TPAD // (NC * NS)
    mesh = plsc.VectorSubcoreMesh(core_axis_name="c", subcore_axis_name="s")

    @functools.partial(
        pl.kernel,
        out_type=[
            jax.ShapeDtypeStruct((TPAD, IN_DIM), jnp.float32),
            jax.ShapeDtypeStruct((TPAD, IN_DIM), jnp.float32),
            jax.ShapeDtypeStruct((TPAD, IN_DIM), jnp.float32),
            jax.ShapeDtypeStruct((TPAD, IN_DIM), jnp.float32),
        ],
        mesh=mesh,
        scratch_types=[
            pltpu.VMEM((rows_w,), jnp.int32),
            pltpu.VMEM((rows_w, IN_DIM), jnp.float32),
            pltpu.SemaphoreType.DMA,
        ],
    )
    def k(f_hbm, m_hbm, ch0_hbm, ch1_hbm, t0_hbm, t1_hbm, c0_hbm, c1_hbm,
          idx_v, rows_v, sem):
        cid = lax.axis_index("c")
        sid = lax.axis_index("s")
        wid = sid * NC + cid
        base = wid * rows_w
        for ch_hbm, t_hbm, c_hbm in ((ch0_hbm, t0_hbm, c0_hbm),
                                     (ch1_hbm, t1_hbm, c1_hbm)):
            pltpu.sync_copy(ch_hbm.at[pl.ds(base, rows_w)], idx_v)
            pltpu.async_copy(f_hbm.at[idx_v], rows_v, sem).wait()
            pltpu.sync_copy(rows_v, t_hbm.at[pl.ds(base, rows_w)])
            pltpu.async_copy(m_hbm.at[idx_v], rows_v, sem).wait()
            pltpu.sync_copy(rows_v, c_hbm.at[pl.ds(base, rows_w)])

    return k(features, ctrmap, ch0, ch1)


def _smote_body(t0_ref, t1_ref, c0_ref, c1_ref, smw_ref, new0_ref, new1_ref):
    smw = smw_ref[...]
    for t_ref, c_ref, o_ref in ((t0_ref, c0_ref, new0_ref),
                                (t1_ref, c1_ref, new1_ref)):
        tail = t_ref[...]
        o_ref[...] = tail + lax.dot_general(
            tail - c_ref[...], smw, (((1,), (0,)), ((), ())),
            precision=_HI, preferred_element_type=jnp.float32)


def _smote(t0, t1, c0, c1, sm_w):
    return pl.pallas_call(
        _smote_body,
        out_shape=[
            jax.ShapeDtypeStruct((TPAD, IN_DIM), jnp.float32),
            jax.ShapeDtypeStruct((TPAD, IN_DIM), jnp.float32),
        ],
    )(t0, t1, c0, c1, sm_w)


def _pad_edges(src, dst):
    s2 = src.reshape(NS, EDGES_PER_TILE)
    sp = jnp.zeros((NS, EPT - EDGES_PER_TILE), jnp.int32)
    d2 = dst.reshape(NS, EDGES_PER_TILE)
    dp = jnp.full((NS, EPT - EDGES_PER_TILE), N_AUG, jnp.int32)
    return (jnp.concatenate([s2, sp], axis=1).reshape(-1),
            jnp.concatenate([d2, dp], axis=1).reshape(-1))


def _scatter_sc(x2, src, dst, zeros, thresh):
    mesh = plsc.VectorSubcoreMesh(core_axis_name="c", subcore_axis_name="s")

    @functools.partial(
        pl.kernel,
        out_type=jax.ShapeDtypeStruct((NC, NPAD, 128), jnp.float32),
        mesh=mesh,
        scratch_types=[
            pltpu.VMEM((EPT,), jnp.int32),
            pltpu.VMEM((EPT,), jnp.int32),
            pltpu.VMEM((IBUF, EB), jnp.int32),
            pltpu.VMEM((IBUF, EB), jnp.int32),
            pltpu.VMEM((NBUF, EB, 128), jnp.float32),
            pltpu.VMEM_SHARED((NPAD, 128), jnp.float32),
            pltpu.SemaphoreType.DMA,
            pltpu.SemaphoreType.DMA,
        ],
    )
    def k(x2_hbm, src_hbm, dst_hbm, z_hbm, out_hbm,
          src_v, dst_v, gidx_v, didx_v, rows_v, acc_sh, gs0, gs1):
        cid = lax.axis_index("c")
        sid = lax.axis_index("s")
        gsems = (gs0, gs1)
        ebase = sid * EPT
        pltpu.sync_copy(src_hbm.at[pl.ds(ebase, EPT)], src_v)
        pltpu.sync_copy(dst_hbm.at[pl.ds(ebase, EPT)], dst_v)

        def compute_idx(j, slot):
            off = j * EB
            for q in range(EB // LANES):
                s = src_v[pl.ds(off + q * LANES, LANES)]
                s = s + jnp.where(s >= thresh, 1000, 0)
                gidx_v[slot, pl.ds(q * LANES, LANES)] = s * 2 + cid
                d = dst_v[pl.ds(off + q * LANES, LANES)]
                didx_v[slot, pl.ds(q * LANES, LANES)] = d

        for b in range(NBUF):
            compute_idx(b, b)
            pltpu.async_copy(x2_hbm.at[gidx_v.at[b]], rows_v.at[b],
                             gsems[b])
        pltpu.sync_copy(z_hbm, acc_sh.at[pl.ds(sid * ROWS_PER_TILE,
                                               ROWS_PER_TILE)])
        plsc.subcore_barrier()

        def body(g, carry):
            for b6 in range(6):
                j = g * 6 + b6
                ib = b6 % IBUF
                ib2 = (b6 + NBUF) % IBUF
                rb = b6 % NBUF
                @pl.when(j + NBUF < NB)
                def _():
                    compute_idx(j + NBUF, ib2)

                pltpu.make_async_copy(
                    x2_hbm.at[gidx_v.at[ib]], rows_v.at[rb],
                    gsems[rb]).wait()
                pltpu.sync_copy(rows_v.at[rb], acc_sh.at[didx_v.at[ib]],
                                add=True)

                @pl.when(j + NBUF < NB)
                def _():
                    pltpu.async_copy(x2_hbm.at[gidx_v.at[ib2]],
                                     rows_v.at[rb], gsems[rb])
            return carry

        lax.fori_loop(0, NB // 6, body, 0)
        plsc.subcore_barrier()
        pltpu.sync_copy(
            acc_sh.at[pl.ds(sid * ROWS_PER_TILE, ROWS_PER_TILE)],
            out_hbm.at[cid, pl.ds(sid * ROWS_PER_TILE, ROWS_PER_TILE)])

    return k(x2, src, dst, zeros)


def _layer_body(relu, a_ref, w_ref, b_ref, o_ref):
    w = w_ref[...]
    h = _dot3(a_ref[0], w[:128]) + _dot3(a_ref[1], w[128:]) + b_ref[...]
    o_ref[...] = jnp.maximum(h, 0.0) if relu else h


def _layer(agg, W, b, relu):
    blk = 1376
    grid = NPAD // blk
    return pl.pallas_call(
        functools.partial(_layer_body, relu),
        grid=(grid,),
        in_specs=[
            pl.BlockSpec((2, blk, 128), lambda i: (0, i, 0)),
            pl.BlockSpec((256, HID_DIM), lambda i: (0, 0)),
            pl.BlockSpec((1, HID_DIM), lambda i: (0, 0)),
        ],
        out_specs=pl.BlockSpec((blk, HID_DIM), lambda i: (i, 0)),
        out_shape=jax.ShapeDtypeStruct((NPAD, HID_DIM), jnp.float32),
    )(agg, W, b[None, :])


def _final_body(a0_ref, a1_ref, w2_ref, b2_ref, saw_ref, sab_ref, saa_ref,
                e0_ref, e1_ref, out_ref):
    w2 = w2_ref[...]
    b2 = b2_ref[...]

    def dense(a_ref):
        return _dot3(a_ref[0], w2[:128]) + _dot3(a_ref[1], w2[128:]) + b2

    e0 = dense(a0_ref)
    e1 = dense(a1_ref)
    e0_ref[...] = e0
    e1_ref[...] = e1
    saw = saw_ref[...]
    sab = sab_ref[...]
    saa = saa_ref[...]

    def score(e):
        t = jnp.tanh(lax.dot_general(e, saw, (((1,), (0,)), ((), ())),
                                     precision=_HI,
                                     preferred_element_type=jnp.float32) + sab)
        return jnp.sum(t * saa, axis=1, keepdims=True)

    w0 = score(e0)
    w1 = score(e1)
    m = jnp.maximum(w0, w1)
    x0 = jnp.exp(w0 - m)
    x1 = jnp.exp(w1 - m)
    inv = 1.0 / (x0 + x1)
    out_ref[...] = (x0 * e0 + x1 * e1) * inv


def _final(agg0, agg1, W2, b2, sa_w, sa_b, sa_a):
    blk = 1376
    grid = NPAD // blk
    return pl.pallas_call(
        _final_body,
        grid=(grid,),
        in_specs=[
            pl.BlockSpec((2, blk, 128), lambda i: (0, i, 0)),
            pl.BlockSpec((2, blk, 128), lambda i: (0, i, 0)),
            pl.BlockSpec((256, HID_DIM), lambda i: (0, 0)),
            pl.BlockSpec((1, HID_DIM), lambda i: (0, 0)),
            pl.BlockSpec((HID_DIM, ATT_DIM), lambda i: (0, 0)),
            pl.BlockSpec((1, ATT_DIM), lambda i: (0, 0)),
            pl.BlockSpec((1, ATT_DIM), lambda i: (0, 0)),
        ],
        out_specs=[
            pl.BlockSpec((blk, HID_DIM), lambda i: (i, 0)),
            pl.BlockSpec((blk, HID_DIM), lambda i: (i, 0)),
            pl.BlockSpec((blk, HID_DIM), lambda i: (i, 0)),
        ],
        out_shape=[
            jax.ShapeDtypeStruct((NPAD, HID_DIM), jnp.float32),
            jax.ShapeDtypeStruct((NPAD, HID_DIM), jnp.float32),
            jax.ShapeDtypeStruct((NPAD, HID_DIM), jnp.float32),
        ],
    )(agg0, agg1, W2, b2[None, :], sa_w, sa_b[None, :], sa_a.reshape(1, ATT_DIM))


def kernel(features, labels, chosen_tails_0, chosen_tails_1, edge_index_0,
           edge_index_1, sm_weight_center, W1, b1, W2, b2, sa_w, sa_b, sa_a):
    i32 = jnp.int32
    labels = labels.astype(i32)
    pad = jnp.zeros((TPAD - N_TAILS,), i32)
    ch0 = jnp.concatenate([chosen_tails_0.astype(i32), pad])
    ch1 = jnp.concatenate([chosen_tails_1.astype(i32), pad])
    src0, dst0 = edge_index_0[0].astype(i32), edge_index_0[1].astype(i32)
    src1, dst1 = edge_index_1[0].astype(i32), edge_index_1[1].astype(i32)
    zeros = jnp.zeros((ROWS_PER_TILE, 128), jnp.float32)

    src0, dst0 = _pad_edges(src0, dst0)
    src1, dst1 = _pad_edges(src1, dst1)

    csum, cnt = _centers(features, labels)
    ctr, ctrmap = _ctrmap(labels, csum, cnt)
    t0, t1, c0, c1 = _tails_sc(features, ctrmap, ch0, ch1)
    new0, new1 = _smote(t0, t1, c0, c1, sm_weight_center)

    x = jnp.concatenate([features, new0[:N_TAILS], new1[:N_TAILS], ctr[:4]],
                        axis=0)
    x2 = x.reshape(-1, 128)

    agg10 = _scatter_sc(x2, src0, dst0, zeros, N_NODES + N_TAILS)
    agg11 = _scatter_sc(x2, src1, dst1, zeros, N_NODES)
    h0 = _layer(agg10, W1, b1, relu=True)
    h1 = _layer(agg11, W1, b1, relu=True)

    big = 2 ** 30
    agg20 = _scatter_sc(h0.reshape(-1, 128), src0, dst0, zeros, big)
    agg21 = _scatter_sc(h1.reshape(-1, 128), src1, dst1, zeros, big)
    e0, e1, outp = _final(agg20, agg21, W2, b2, sa_w, sa_b, sa_a)

    return (e0[:N_AUG], e1[:N_AUG], outp[:N_NODES])
